# baseline (speedup 1.0000x reference)
import functools
import math

import jax
import jax.numpy as jnp
from jax.experimental import pallas as pl
from jax.experimental.pallas import tpu as pltpu

_B, _M, _K, _D, _H, _DFF = 256, 100, 8, 512, 8, 1024
_S = _M + 1
_DH = _D // _H
_BB = 16


def _ln_rows(x2, s, b):
    mu = jnp.mean(x2, axis=1, keepdims=True)
    xc = x2 - mu
    var = jnp.mean(xc * xc, axis=1, keepdims=True)
    return xc * jax.lax.rsqrt(var + 1e-5) * s + b


def _block(x_ref, wq_ref, bq_ref, wk_ref, bk_ref, wv_ref, bv_ref, wo_ref,
           bo_ref, ln1s_ref, ln1b_ref, w1_ref, b1_ref, w2_ref, b2_ref,
           ln2s_ref, ln2b_ref, idx_ref, out_ref, aw_ref):
    n = _BB * _S
    bf = jnp.bfloat16
    f32 = jnp.float32
    xb = x_ref[...]
    x2 = xb.reshape(n, _D)
    xr = _ln_rows(x2, ln1s_ref[...], ln1b_ref[...]).astype(bf)

    q = (jnp.dot(xr, wq_ref[...], preferred_element_type=f32)
         + bq_ref[...]).astype(bf)
    k = (jnp.dot(xr, wk_ref[...], preferred_element_type=f32)
         + bk_ref[...]).astype(bf)
    v = jnp.dot(xr, wv_ref[...], preferred_element_type=f32) + bv_ref[...]

    qh = q.reshape(_BB, _S, _H, _DH).transpose(0, 2, 1, 3).reshape(_BB * _H, _S, _DH)
    kh = k.reshape(_BB, _S, _H, _DH).transpose(0, 2, 1, 3).reshape(_BB * _H, _S, _DH)
    vh = v.reshape(_BB, _S, _H, _DH).transpose(0, 2, 1, 3).reshape(_BB * _H, _S, _DH)

    sd = jax.lax.dot_general(
        qh, kh, (((2,), (2,)), ((0,), (0,))),
        preferred_element_type=f32).astype(bf)

    idxv = idx_ref[...]
    tio = jax.lax.broadcasted_iota(jnp.int32, (_S, _K, _S), 2)
    ef = (idxv[:, :, None] == tio).astype(bf)
    m = jnp.sum(ef, axis=1, dtype=f32).astype(bf)

    ex = jnp.exp(sd)
    em = m[None] * ex

    vh_aug = jnp.concatenate(
        [vh, jnp.ones((_BB * _H, _S, 1), f32)], axis=2).astype(bf)
    ctx_aug = jax.lax.dot_general(
        em, vh_aug, (((2,), (1,)), ((0,), (0,))),
        preferred_element_type=f32)
    z = ctx_aug[:, :, _DH:_DH + 1]
    rn = 1.0 / z
    ctx = ctx_aug[:, :, :_DH] * rn

    ctx2 = ctx.astype(bf).reshape(_BB, _H, _S, _DH).transpose(0, 2, 1, 3).reshape(n, _D)

    ao = jnp.dot(ctx2, wo_ref[...], preferred_element_type=f32) + bo_ref[...]
    t = x2 + ao

    xr2 = _ln_rows(t, ln2s_ref[...], ln2b_ref[...]).astype(bf)
    h1 = jnp.maximum(
        jnp.dot(xr2, w1_ref[...], preferred_element_type=f32) + b1_ref[...],
        0.0).astype(bf)
    out = t + jnp.dot(h1, w2_ref[...], preferred_element_type=f32) + b2_ref[...]
    out_ref[...] = out.reshape(_BB, _S, _D)

    exn = ex * rn.astype(bf)
    pm = jnp.sum(exn.reshape(_BB, _H, _S, _S), axis=1)
    aw = jnp.sum(ef[None] * pm[:, :, None, :], axis=3,
                 dtype=f32) * jnp.float32(1.0 / _H)
    aw_ref[...] = aw


@jax.jit
def kernel(x, Wq, bq, Wk, bk, Wv, bv, Wo, bo, ln1_s, ln1_b, W1, b1, W2, b2,
           ln2_s, ln2_b, idx):
    bf = jnp.bfloat16
    vec = lambda a: a.reshape(1, -1)
    sc = 1.0 / math.sqrt(_DH)
    wq = (Wq * sc).astype(bf)
    bqv = vec(bq * sc).astype(bf)
    wk = Wk.astype(bf)
    bkv = vec(bk).astype(bf)
    args = (x, wq, bqv, wk, bkv, Wv.astype(bf), vec(bv), Wo.astype(bf),
            vec(bo), vec(ln1_s), vec(ln1_b), W1.astype(bf), vec(b1),
            W2.astype(bf), vec(b2), vec(ln2_s), vec(ln2_b), idx)
    grid = (_B // _BB,)
    full2 = lambda a: pl.BlockSpec(a.shape, lambda i: (0, 0))
    out, aw = pl.pallas_call(
        _block,
        grid=grid,
        in_specs=[pl.BlockSpec((_BB, _S, _D), lambda i: (i, 0, 0))]
        + [full2(a) for a in args[1:]],
        out_specs=[
            pl.BlockSpec((_BB, _S, _D), lambda i: (i, 0, 0)),
            pl.BlockSpec((_BB, _S, _K), lambda i: (i, 0, 0)),
        ],
        out_shape=[
            jax.ShapeDtypeStruct((_B, _S, _D), jnp.float32),
            jax.ShapeDtypeStruct((_B, _S, _K), jnp.float32),
        ],
        compiler_params=pltpu.CompilerParams(
            dimension_semantics=("parallel",)),
    )(*args)
    return (out, aw)


# speedup vs baseline: 1.0737x; 1.0737x over previous
import functools
import math

import jax
import jax.numpy as jnp
from jax.experimental import pallas as pl
from jax.experimental.pallas import tpu as pltpu

_B, _M, _K, _D, _H, _DFF = 256, 100, 8, 512, 8, 1024
_S = _M + 1
_DH = _D // _H
_BB = 8


def _ln_rows(x2, s, b):
    mu = jnp.mean(x2, axis=1, keepdims=True)
    xc = x2 - mu
    var = jnp.mean(xc * xc, axis=1, keepdims=True)
    return xc * jax.lax.rsqrt(var + 1e-5) * s + b


def _block(x_ref, wq_ref, bq_ref, wk_ref, bk_ref, wv_ref, bv_ref, wo_ref,
           bo_ref, ln1s_ref, ln1b_ref, w1_ref, b1_ref, w2_ref, b2_ref,
           ln2s_ref, ln2b_ref, idx_ref, out_ref, aw_ref):
    n = _BB * _S
    bf = jnp.bfloat16
    f32 = jnp.float32
    xb = x_ref[...]
    x2 = xb.reshape(n, _D)
    xr = _ln_rows(x2, ln1s_ref[...], ln1b_ref[...]).astype(bf)

    q = jnp.dot(xr, wq_ref[...], preferred_element_type=f32) + bq_ref[...]
    k = jnp.dot(xr, wk_ref[...], preferred_element_type=f32) + bk_ref[...]
    v = jnp.dot(xr, wv_ref[...], preferred_element_type=f32) + bv_ref[...]

    qh = q.reshape(_BB, _S, _H, _DH).transpose(0, 2, 1, 3).reshape(_BB * _H, _S, _DH)
    kh = k.reshape(_BB, _S, _H, _DH).transpose(0, 2, 1, 3).reshape(_BB * _H, _S, _DH)
    vh = v.reshape(_BB, _S, _H, _DH).transpose(0, 2, 1, 3).reshape(_BB * _H, _S, _DH)

    sd = jax.lax.dot_general(
        qh, kh, (((2,), (2,)), ((0,), (0,))),
        preferred_element_type=f32)

    idxv = idx_ref[...]
    tio = jax.lax.broadcasted_iota(jnp.int32, (_S, _K, _S), 2)
    ef = (idxv[:, :, None] == tio).astype(f32)
    m = jnp.sum(ef, axis=1)

    ex = jnp.exp(sd)
    em = m[None] * ex

    vh_aug = jnp.concatenate(
        [vh, jnp.ones((_BB * _H, _S, 1), f32)], axis=2)
    ctx_aug = jax.lax.dot_general(
        em, vh_aug, (((2,), (1,)), ((0,), (0,))),
        preferred_element_type=f32)
    z = ctx_aug[:, :, _DH:_DH + 1]
    rn = 1.0 / z
    ctx = ctx_aug[:, :, :_DH] * rn

    ctx2 = ctx.astype(bf).reshape(_BB, _H, _S, _DH).transpose(0, 2, 1, 3).reshape(n, _D)

    ao = jnp.dot(ctx2, wo_ref[...], preferred_element_type=f32) + bo_ref[...]
    t = x2 + ao

    xr2 = _ln_rows(t, ln2s_ref[...], ln2b_ref[...]).astype(bf)
    h1 = jnp.maximum(
        jnp.dot(xr2, w1_ref[...], preferred_element_type=f32) + b1_ref[...],
        0.0).astype(bf)
    out = t + jnp.dot(h1, w2_ref[...], preferred_element_type=f32) + b2_ref[...]
    out_ref[...] = out.reshape(_BB, _S, _D)

    exn = ex * rn
    pm = jnp.sum(exn.reshape(_BB, _H, _S, _S), axis=1)
    aw = jnp.sum(ef[None] * pm[:, :, None, :], axis=3) * jnp.float32(1.0 / _H)
    aw_ref[...] = aw


@jax.jit
def kernel(x, Wq, bq, Wk, bk, Wv, bv, Wo, bo, ln1_s, ln1_b, W1, b1, W2, b2,
           ln2_s, ln2_b, idx):
    bf = jnp.bfloat16
    vec = lambda a: a.reshape(1, -1)
    sc = 1.0 / math.sqrt(_DH)
    wq = (Wq * sc).astype(bf)
    bqv = vec(bq * sc).astype(bf)
    wk = Wk.astype(bf)
    bkv = vec(bk).astype(bf)
    args = (x, wq, bqv, wk, bkv, Wv.astype(bf), vec(bv), Wo.astype(bf),
            vec(bo), vec(ln1_s), vec(ln1_b), W1.astype(bf), vec(b1),
            W2.astype(bf), vec(b2), vec(ln2_s), vec(ln2_b), idx)
    grid = (_B // _BB,)
    full2 = lambda a: pl.BlockSpec(a.shape, lambda i: (0, 0))
    out, aw = pl.pallas_call(
        _block,
        grid=grid,
        in_specs=[pl.BlockSpec((_BB, _S, _D), lambda i: (i, 0, 0))]
        + [full2(a) for a in args[1:]],
        out_specs=[
            pl.BlockSpec((_BB, _S, _D), lambda i: (i, 0, 0)),
            pl.BlockSpec((_BB, _S, _K), lambda i: (i, 0, 0)),
        ],
        out_shape=[
            jax.ShapeDtypeStruct((_B, _S, _D), jnp.float32),
            jax.ShapeDtypeStruct((_B, _S, _K), jnp.float32),
        ],
        compiler_params=pltpu.CompilerParams(
            dimension_semantics=("parallel",)),
    )(*args)
    return (out, aw)


# speedup vs baseline: 1.5614x; 1.4542x over previous
import functools
import math

import jax
import jax.numpy as jnp
from jax.experimental import pallas as pl
from jax.experimental.pallas import tpu as pltpu

_B, _M, _K, _D, _H, _DFF = 256, 100, 8, 512, 8, 1024
_S = _M + 1
_S2 = 104
_DH = _D // _H
_BB = 8


def _ln_rows(x2, s, b):
    mu = jnp.mean(x2, axis=1, keepdims=True)
    xc = x2 - mu
    var = jnp.mean(xc * xc, axis=1, keepdims=True)
    return xc * jax.lax.rsqrt(var + 1e-5) * s + b


def _block(x_ref, wq_ref, bq_ref, wk_ref, bk_ref, wv_ref, bv_ref, wo_ref,
           bo_ref, ln1s_ref, ln1b_ref, w1_ref, b1_ref, w2_ref, b2_ref,
           ln2s_ref, ln2b_ref, idx_ref, out_ref, aw_ref):
    bf = jnp.bfloat16
    f32 = jnp.float32
    n = _BB * _S2
    xb = x_ref[...]
    xp = jnp.concatenate([xb, jnp.zeros((_BB, _S2 - _S, _D), f32)], axis=1)
    x2 = xp.reshape(n, _D)
    xr = _ln_rows(x2, ln1s_ref[...], ln1b_ref[...]).astype(bf)

    q = jnp.dot(xr, wq_ref[...], preferred_element_type=f32) + bq_ref[...]
    k = jnp.dot(xr, wk_ref[...], preferred_element_type=f32) + bk_ref[...]
    v = jnp.dot(xr, wv_ref[...], preferred_element_type=f32) + bv_ref[...]
    q3 = q.reshape(_BB, _S2, _D)
    k3 = k.reshape(_BB, _S2, _D)
    v3 = v.reshape(_BB, _S2, _D)

    idxv = idx_ref[...]
    tio = jax.lax.broadcasted_iota(jnp.int32, (_S2, _K, _S2), 2)
    ef = (idxv[:, :, None] == tio).astype(f32)
    m = jnp.sum(ef, axis=1)
    ones_col = jnp.ones((_BB, _S2, 1), f32)

    ctx_parts = []
    pm = jnp.zeros((_BB, _S2, _S2), f32)
    for h in range(_H):
        qh = q3[:, :, h * _DH:(h + 1) * _DH]
        kh = k3[:, :, h * _DH:(h + 1) * _DH]
        vh = v3[:, :, h * _DH:(h + 1) * _DH]
        sdh = jax.lax.dot_general(
            qh, kh, (((2,), (2,)), ((0,), (0,))),
            preferred_element_type=f32)
        exh = jnp.exp(sdh)
        emh = m[None] * exh
        vh_aug = jnp.concatenate([vh, ones_col], axis=2)
        ctxh = jax.lax.dot_general(
            emh, vh_aug, (((2,), (1,)), ((0,), (0,))),
            preferred_element_type=f32)
        rn = 1.0 / ctxh[:, :, _DH:_DH + 1]
        ctx_parts.append(ctxh[:, :, :_DH] * rn)
        pm = pm + exh * rn

    ctx2 = jnp.concatenate(ctx_parts, axis=2).reshape(n, _D).astype(bf)

    ao = jnp.dot(ctx2, wo_ref[...], preferred_element_type=f32) + bo_ref[...]
    t = x2 + ao

    xr2 = _ln_rows(t, ln2s_ref[...], ln2b_ref[...]).astype(bf)
    h1 = jnp.maximum(
        jnp.dot(xr2, w1_ref[...], preferred_element_type=f32) + b1_ref[...],
        0.0).astype(bf)
    out = t + jnp.dot(h1, w2_ref[...], preferred_element_type=f32) + b2_ref[...]
    out_ref[...] = out.reshape(_BB, _S2, _D)[:, :_S, :]

    aw = jnp.sum(ef[None] * pm[:, :, None, :], axis=3) * jnp.float32(1.0 / _H)
    aw_ref[...] = aw[:, :_S, :]


@jax.jit
def kernel(x, Wq, bq, Wk, bk, Wv, bv, Wo, bo, ln1_s, ln1_b, W1, b1, W2, b2,
           ln2_s, ln2_b, idx):
    bf = jnp.bfloat16
    vec = lambda a: a.reshape(1, -1)
    sc = 1.0 / math.sqrt(_DH)
    wq = (Wq * sc).astype(bf)
    bqv = vec(bq * sc).astype(bf)
    wk = Wk.astype(bf)
    bkv = vec(bk).astype(bf)
    idxp = jnp.pad(idx, ((0, _S2 - _S), (0, 0)))
    args = (x, wq, bqv, wk, bkv, Wv.astype(bf), vec(bv), Wo.astype(bf),
            vec(bo), vec(ln1_s), vec(ln1_b), W1.astype(bf), vec(b1),
            W2.astype(bf), vec(b2), vec(ln2_s), vec(ln2_b), idxp)
    grid = (_B // _BB,)
    full2 = lambda a: pl.BlockSpec(a.shape, lambda i: (0, 0))
    out, aw = pl.pallas_call(
        _block,
        grid=grid,
        in_specs=[pl.BlockSpec((_BB, _S, _D), lambda i: (i, 0, 0))]
        + [full2(a) for a in args[1:]],
        out_specs=[
            pl.BlockSpec((_BB, _S, _D), lambda i: (i, 0, 0)),
            pl.BlockSpec((_BB, _S, _K), lambda i: (i, 0, 0)),
        ],
        out_shape=[
            jax.ShapeDtypeStruct((_B, _S, _D), jnp.float32),
            jax.ShapeDtypeStruct((_B, _S, _K), jnp.float32),
        ],
        compiler_params=pltpu.CompilerParams(
            dimension_semantics=("parallel",)),
    )(*args)
    return (out, aw)
